# baseline (speedup 1.0000x reference)
import numpy as np
import jax
import jax.numpy as jnp
from jax import lax
from jax.experimental import pallas as pl
from jax.experimental.pallas import tpu as pltpu

_IMG = 224
_FMAP = 26
_SIGMA = 4.0
_NQ = 676
_MP = 704
_BN = 1024
_DIMS = (((1,), (1,)), ((), ()))


def _resize_blur_matrix() -> np.ndarray:
    scale = _IMG / _FMAP
    x = (np.arange(_IMG) + 0.5) / scale - 0.5
    j = np.arange(_FMAP)
    w = np.maximum(0.0, 1.0 - np.abs(x[:, None] - j[None, :]))
    w = w / w.sum(axis=1, keepdims=True)
    ks = 2 * int(4.0 * _SIGMA) + 1
    r = ks // 2
    t = np.arange(ks) - r
    k = np.exp(-0.5 * (t / _SIGMA) ** 2)
    k = k / k.sum()
    g = np.zeros((_IMG, _IMG))
    for i in range(_IMG):
        for dt in range(ks):
            p = i + dt - r
            if p < 0:
                p = -p
            if p > _IMG - 1:
                p = 2 * (_IMG - 1) - p
            g[i, p] += k[dt]
    return (g @ w).astype(np.float32)


_A = _resize_blur_matrix()
_AT = np.ascontiguousarray(_A.T)


def _split(x):
    hi = x.astype(jnp.bfloat16)
    lo = (x - hi.astype(jnp.float32)).astype(jnp.bfloat16)
    return hi, lo


def _dot3(xh, xl, yh, yl):
    kw = dict(preferred_element_type=jnp.float32)
    return (lax.dot_general(xh, yh, _DIMS, **kw)
            + (lax.dot_general(xh, yl, _DIMS, **kw)
               + lax.dot_general(xl, yh, _DIMS, **kw)))


def _row_sumsq(x):
    eh, el = _split(x * x)
    ones = jnp.ones((1, x.shape[1]), jnp.bfloat16)
    kw = dict(preferred_element_type=jnp.float32)
    return (lax.dot_general(ones, eh, _DIMS, **kw)
            + lax.dot_general(ones, el, _DIMS, **kw))


def _k1_body(patch_ref, ah_ref, al_ref, lib_ref,
             minval_ref, sstar_ref, sidx_ref, staridx_ref,
             curmin_ref, curidx_ref):
    j = pl.program_id(0)
    nb = pl.num_programs(0)
    b = lib_ref[...]
    bh, bl = _split(b)
    ab = _dot3(bh, bl, ah_ref[...], al_ref[...])
    b2 = jnp.sum(b * b, axis=1, keepdims=True)
    d2 = b2 - 2.0 * ab
    bmin = jnp.min(d2, axis=0, keepdims=True)
    rows = lax.broadcasted_iota(jnp.int32, d2.shape, 0)
    bidx = jnp.min(jnp.where(d2 == bmin, rows, d2.shape[0]),
                   axis=0, keepdims=True)

    @pl.when(j == 0)
    def _init():
        curmin_ref[...] = jnp.full(curmin_ref.shape, jnp.inf, jnp.float32)
        curidx_ref[...] = jnp.zeros(curidx_ref.shape, jnp.int32)

    cm = curmin_ref[...]
    better = bmin < cm
    curmin_ref[...] = jnp.where(better, bmin, cm)
    curidx_ref[...] = jnp.where(better, j * _BN + bidx, curidx_ref[...])

    @pl.when(j == nb - 1)
    def _fin():
        a2 = _row_sumsq(patch_ref[...])
        dist = jnp.sqrt(jnp.maximum(curmin_ref[...] + a2, 1e-12))
        minval_ref[...] = dist
        cols = lax.broadcasted_iota(jnp.int32, dist.shape, 1)
        mv = jnp.where(cols < _NQ, dist, -1.0)
        smax = jnp.max(mv)
        sidx = jnp.min(jnp.where(mv == smax, cols, _MP))
        sstar_ref[0, 0] = smax
        sidx_ref[0, 0] = sidx
        staridx_ref[0, 0] = jnp.sum(
            jnp.where(cols == sidx, curidx_ref[...], 0))


def _k2_body(m8_ref, mh_ref, ml_ref, sstar_ref, lib_ref, s_ref, d2all_ref):
    j = pl.program_id(0)
    nb = pl.num_programs(0)
    b = lib_ref[...]
    bh, bl = _split(b)
    ab = _dot3(mh_ref[...], ml_ref[...], bh, bl)
    b2 = _row_sumsq(b)
    d2all_ref[:, pl.ds(j * _BN, _BN)] = b2 - 2.0 * ab

    @pl.when(j == nb - 1)
    def _fin():
        m = m8_ref[...]
        n = d2all_ref.shape[1]
        m2 = jnp.sum(m * m, axis=1, keepdims=True)
        all_d2 = d2all_ref[...] + m2
        star = all_d2[0:1, :]
        test = all_d2[1:2, :]
        cols = lax.broadcasted_iota(jnp.int32, star.shape, 1)
        inf = jnp.float32(jnp.inf)
        i0 = jnp.min(jnp.where(star == jnp.min(star), cols, n))
        star1 = jnp.where(cols == i0, inf, star)
        i1 = jnp.min(jnp.where(star1 == jnp.min(star1), cols, n))
        star2 = jnp.where(cols == i1, inf, star1)
        i2 = jnp.min(jnp.where(star2 == jnp.min(star2), cols, n))
        dk1 = jnp.sqrt(jnp.maximum(
            jnp.sum(jnp.where(cols == i1, test, 0.0)), 0.0))
        dk2 = jnp.sqrt(jnp.maximum(
            jnp.sum(jnp.where(cols == i2, test, 0.0)), 0.0))
        dcap = jnp.sqrt(jnp.float32(m.shape[1]))
        sstar = sstar_ref[0, 0]
        w = 1.0 - jnp.exp(sstar / dcap) / (jnp.exp(dk1 / dcap)
                                           + jnp.exp(dk2 / dcap))
        s_ref[0, 0] = w * sstar


def _k3_body(s_ref, a_ref, at_ref, out_ref):
    mm = (((1,), (0,)), ((), ()))
    t = lax.dot_general(a_ref[...], s_ref[...], mm,
                        preferred_element_type=jnp.float32)
    out_ref[...] = lax.dot_general(t, at_ref[...], mm,
                                   preferred_element_type=jnp.float32)


def kernel(patch, patch_lib):
    n, d = patch_lib.shape
    nb = n // _BN
    patch_p = jnp.pad(patch, ((0, _MP - patch.shape[0]), (0, 0)))
    ah = patch_p.astype(jnp.bfloat16)
    al = (patch_p - ah.astype(jnp.float32)).astype(jnp.bfloat16)

    minval, sstar, sidx, staridx = pl.pallas_call(
        _k1_body,
        grid=(nb,),
        in_specs=[
            pl.BlockSpec((_MP, d), lambda j: (0, 0)),
            pl.BlockSpec((_MP, d), lambda j: (0, 0)),
            pl.BlockSpec((_MP, d), lambda j: (0, 0)),
            pl.BlockSpec((_BN, d), lambda j: (j, 0)),
        ],
        out_specs=[
            pl.BlockSpec((1, _MP), lambda j: (0, 0)),
            pl.BlockSpec(memory_space=pltpu.SMEM),
            pl.BlockSpec(memory_space=pltpu.SMEM),
            pl.BlockSpec(memory_space=pltpu.SMEM),
        ],
        out_shape=[
            jax.ShapeDtypeStruct((1, _MP), jnp.float32),
            jax.ShapeDtypeStruct((1, 1), jnp.float32),
            jax.ShapeDtypeStruct((1, 1), jnp.int32),
            jax.ShapeDtypeStruct((1, 1), jnp.int32),
        ],
        scratch_shapes=[
            pltpu.VMEM((1, _MP), jnp.float32),
            pltpu.VMEM((1, _MP), jnp.int32),
        ],
    )(patch_p, ah, al, patch_lib)

    m_star = lax.dynamic_index_in_dim(patch_lib, staridx[0, 0], axis=0)
    m_test = lax.dynamic_index_in_dim(patch, sidx[0, 0], axis=0)
    m8 = jnp.zeros((8, d), jnp.float32)
    m8 = lax.dynamic_update_slice(m8, m_star, (0, 0))
    m8 = lax.dynamic_update_slice(m8, m_test, (1, 0))
    mh = m8.astype(jnp.bfloat16)
    ml = (m8 - mh.astype(jnp.float32)).astype(jnp.bfloat16)

    s = pl.pallas_call(
        _k2_body,
        grid=(nb,),
        in_specs=[
            pl.BlockSpec((8, d), lambda j: (0, 0)),
            pl.BlockSpec((8, d), lambda j: (0, 0)),
            pl.BlockSpec((8, d), lambda j: (0, 0)),
            pl.BlockSpec(memory_space=pltpu.SMEM),
            pl.BlockSpec((_BN, d), lambda j: (j, 0)),
        ],
        out_specs=pl.BlockSpec(memory_space=pltpu.SMEM),
        out_shape=jax.ShapeDtypeStruct((1, 1), jnp.float32),
        scratch_shapes=[pltpu.VMEM((8, n), jnp.float32)],
    )(m8, mh, ml, sstar, patch_lib)

    smat = minval[0, :_NQ].reshape(_FMAP, _FMAP)
    smap = pl.pallas_call(
        _k3_body,
        in_specs=[
            pl.BlockSpec((_FMAP, _FMAP), lambda: (0, 0)),
            pl.BlockSpec((_IMG, _FMAP), lambda: (0, 0)),
            pl.BlockSpec((_FMAP, _IMG), lambda: (0, 0)),
        ],
        out_specs=pl.BlockSpec((_IMG, _IMG), lambda: (0, 0)),
        out_shape=jax.ShapeDtypeStruct((_IMG, _IMG), jnp.float32),
    )(smat, jnp.asarray(_A), jnp.asarray(_AT))

    return s.reshape(()), smap.reshape(1, 1, _IMG, _IMG)


# speedup vs baseline: 1.2280x; 1.2280x over previous
import numpy as np
import jax
import jax.numpy as jnp
from jax import lax
from jax.experimental import pallas as pl
from jax.experimental.pallas import tpu as pltpu

_IMG = 224
_FMAP = 26
_SIGMA = 4.0
_NQ = 676
_MP = 704
_BN = 1024
_NC = 16
_DIMS = (((1,), (1,)), ((), ()))


def _resize_blur_matrix() -> np.ndarray:
    scale = _IMG / _FMAP
    x = (np.arange(_IMG) + 0.5) / scale - 0.5
    j = np.arange(_FMAP)
    w = np.maximum(0.0, 1.0 - np.abs(x[:, None] - j[None, :]))
    w = w / w.sum(axis=1, keepdims=True)
    ks = 2 * int(4.0 * _SIGMA) + 1
    r = ks // 2
    t = np.arange(ks) - r
    k = np.exp(-0.5 * (t / _SIGMA) ** 2)
    k = k / k.sum()
    g = np.zeros((_IMG, _IMG))
    for i in range(_IMG):
        for dt in range(ks):
            p = i + dt - r
            if p < 0:
                p = -p
            if p > _IMG - 1:
                p = 2 * (_IMG - 1) - p
            g[i, p] += k[dt]
    return (g @ w).astype(np.float32)


_A = _resize_blur_matrix()
_B = np.zeros((_IMG, _MP), np.float32)
_C = np.zeros((_IMG, _MP), np.float32)
_B[:, :_NQ] = _A[:, np.arange(_NQ) // _FMAP]
_C[:, :_NQ] = _A[:, np.arange(_NQ) % _FMAP]
_BF16 = np.dtype(jnp.bfloat16.dtype)
_CH = _C.astype(_BF16)
_CL = (_C - _CH.astype(np.float32)).astype(_BF16)


def _split(x):
    hi = x.astype(jnp.bfloat16)
    lo = (x - hi.astype(jnp.float32)).astype(jnp.bfloat16)
    return hi, lo


def _dot3(xh, xl, yh, yl):
    kw = dict(preferred_element_type=jnp.float32)
    return (lax.dot_general(xh, yh, _DIMS, **kw)
            + (lax.dot_general(xh, yl, _DIMS, **kw)
               + lax.dot_general(xl, yh, _DIMS, **kw)))


def _row_sumsq(x):
    eh, el = _split(x * x)
    ones = jnp.ones((1, x.shape[1]), jnp.bfloat16)
    kw = dict(preferred_element_type=jnp.float32)
    return (lax.dot_general(ones, eh, _DIMS, **kw)
            + lax.dot_general(ones, el, _DIMS, **kw))


def _k1_body(patch_ref, ahs_ref, lib_ref,
             minval_ref, cand_ref, a16_ref, curmin_ref):
    j = pl.program_id(0)
    nb = pl.num_programs(0)
    b = lib_ref[...]
    ab = lax.dot_general(b.astype(jnp.bfloat16), ahs_ref[...], _DIMS,
                         preferred_element_type=jnp.float32)
    b2 = jnp.sum(b * b, axis=1, keepdims=True)
    bmin = jnp.min(ab + b2, axis=0, keepdims=True)

    @pl.when(j == 0)
    def _init():
        curmin_ref[...] = jnp.full(curmin_ref.shape, jnp.inf, jnp.float32)

    curmin_ref[...] = jnp.minimum(curmin_ref[...], bmin)

    @pl.when(j == nb - 1)
    def _fin():
        a2 = _row_sumsq(patch_ref[...])
        dist = jnp.sqrt(jnp.maximum(curmin_ref[...] + a2, 1e-12))
        minval_ref[...] = dist
        cols = lax.broadcasted_iota(jnp.int32, dist.shape, 1)
        mv = jnp.where(cols < _NQ, dist, -1.0)
        rows_a = lax.broadcasted_iota(jnp.int32, patch_ref.shape, 0)
        cols16 = lax.broadcasted_iota(jnp.int32, (1, _NC), 1)
        candvec = jnp.full((1, _NC), _MP, jnp.int32)
        for k in range(_NC):
            vmax = jnp.max(mv)
            idx = jnp.min(jnp.where(mv == vmax, cols, _MP))
            candvec = jnp.where(cols16 == k, idx, candvec)
            mv = jnp.where(cols == idx, -1.0, mv)
            a16_ref[k:k + 1, :] = jnp.sum(
                jnp.where(rows_a == idx, patch_ref[...], 0.0),
                axis=0, keepdims=True)
        cand_ref[...] = candvec


def _k1b_body(a16_ref, cand_ref, lib_ref,
              sstar_ref, sidx_ref, staridx_ref, mtest_ref,
              curmin_ref, curidx_ref):
    j = pl.program_id(0)
    nb = pl.num_programs(0)
    a16 = a16_ref[...]
    ah, al = _split(a16)
    b = lib_ref[...]
    bh, bl = _split(b)
    ab = _dot3(bh, bl, ah, al)
    b2 = jnp.sum(b * b, axis=1, keepdims=True)
    d2 = b2 - 2.0 * ab
    bmin = jnp.min(d2, axis=0, keepdims=True)
    rows = lax.broadcasted_iota(jnp.int32, d2.shape, 0)
    bidx = jnp.min(jnp.where(d2 == bmin, rows, d2.shape[0]),
                   axis=0, keepdims=True)

    @pl.when(j == 0)
    def _init():
        curmin_ref[...] = jnp.full(curmin_ref.shape, jnp.inf, jnp.float32)
        curidx_ref[...] = jnp.zeros(curidx_ref.shape, jnp.int32)

    cm = curmin_ref[...]
    better = bmin < cm
    curmin_ref[...] = jnp.where(better, bmin, cm)
    curidx_ref[...] = jnp.where(better, j * _BN + bidx, curidx_ref[...])

    @pl.when(j == nb - 1)
    def _fin():
        a2 = _row_sumsq(a16)
        dist = jnp.sqrt(jnp.maximum(curmin_ref[...] + a2, 1e-12))
        cand = cand_ref[...]
        cols = lax.broadcasted_iota(jnp.int32, dist.shape, 1)
        smax = jnp.max(dist)
        sidx = jnp.min(jnp.where(dist == smax, cand, jnp.int32(2 ** 30)))
        lpos = jnp.min(jnp.where(cand == sidx, cols, _NC))
        sstar_ref[0, 0] = smax
        sidx_ref[0, 0] = sidx
        staridx_ref[0, 0] = jnp.sum(
            jnp.where(cols == lpos, curidx_ref[...], 0))
        rows_a = lax.broadcasted_iota(jnp.int32, a16.shape, 0)
        mtest_ref[...] = jnp.sum(
            jnp.where(rows_a == lpos, a16, 0.0), axis=0, keepdims=True)


def _k2_body(mstar_ref, mtest_ref, sstar_ref, minval_ref, b_ref, ch_ref,
             cl_ref, lib_ref, s_ref, smap_ref, d2all_ref):
    j = pl.program_id(0)
    nb = pl.num_programs(0)
    m = jnp.concatenate(
        [mstar_ref[...], mtest_ref[...],
         jnp.zeros((6, mstar_ref.shape[1]), jnp.float32)], axis=0)
    mh, ml = _split(m)
    b = lib_ref[...]
    bh, bl = _split(b)
    ab = _dot3(mh, ml, bh, bl)
    b2 = _row_sumsq(b)
    d2all_ref[:, pl.ds(j * _BN, _BN)] = b2 - 2.0 * ab

    @pl.when(j == nb - 1)
    def _fin():
        n = d2all_ref.shape[1]
        m2 = jnp.sum(m * m, axis=1, keepdims=True)
        all_d2 = d2all_ref[...] + m2
        star = all_d2[0:1, :]
        test = all_d2[1:2, :]
        cols = lax.broadcasted_iota(jnp.int32, star.shape, 1)
        inf = jnp.float32(jnp.inf)
        i0 = jnp.min(jnp.where(star == jnp.min(star), cols, n))
        star1 = jnp.where(cols == i0, inf, star)
        i1 = jnp.min(jnp.where(star1 == jnp.min(star1), cols, n))
        star2 = jnp.where(cols == i1, inf, star1)
        i2 = jnp.min(jnp.where(star2 == jnp.min(star2), cols, n))
        dk1 = jnp.sqrt(jnp.maximum(
            jnp.sum(jnp.where(cols == i1, test, 0.0)), 0.0))
        dk2 = jnp.sqrt(jnp.maximum(
            jnp.sum(jnp.where(cols == i2, test, 0.0)), 0.0))
        dcap = jnp.sqrt(jnp.float32(m.shape[1]))
        sstar = sstar_ref[0, 0]
        w = 1.0 - jnp.exp(sstar / dcap) / (jnp.exp(dk1 / dcap)
                                           + jnp.exp(dk2 / dcap))
        s_ref[0, 0] = w * sstar
        t = b_ref[...] * minval_ref[...]
        th, tl = _split(t)
        smap_ref[...] = _dot3(th, tl, ch_ref[...], cl_ref[...])


def kernel(patch, patch_lib):
    n, d = patch_lib.shape
    nb = n // _BN
    patch_p = jnp.pad(patch, ((0, _MP - patch.shape[0]), (0, 0)))
    ahs = (-2.0 * patch_p).astype(jnp.bfloat16)

    minval, cand, a16 = pl.pallas_call(
        _k1_body,
        grid=(nb,),
        in_specs=[
            pl.BlockSpec((_MP, d), lambda j: (0, 0)),
            pl.BlockSpec((_MP, d), lambda j: (0, 0)),
            pl.BlockSpec((_BN, d), lambda j: (j, 0)),
        ],
        out_specs=[
            pl.BlockSpec((1, _MP), lambda j: (0, 0)),
            pl.BlockSpec((1, _NC), lambda j: (0, 0)),
            pl.BlockSpec((_NC, d), lambda j: (0, 0)),
        ],
        out_shape=[
            jax.ShapeDtypeStruct((1, _MP), jnp.float32),
            jax.ShapeDtypeStruct((1, _NC), jnp.int32),
            jax.ShapeDtypeStruct((_NC, d), jnp.float32),
        ],
        scratch_shapes=[pltpu.VMEM((1, _MP), jnp.float32)],
    )(patch_p, ahs, patch_lib)

    sstar, sidx, staridx, mtest = pl.pallas_call(
        _k1b_body,
        grid=(nb,),
        in_specs=[
            pl.BlockSpec((_NC, d), lambda j: (0, 0)),
            pl.BlockSpec((1, _NC), lambda j: (0, 0)),
            pl.BlockSpec((_BN, d), lambda j: (j, 0)),
        ],
        out_specs=[
            pl.BlockSpec(memory_space=pltpu.SMEM),
            pl.BlockSpec(memory_space=pltpu.SMEM),
            pl.BlockSpec(memory_space=pltpu.SMEM),
            pl.BlockSpec((1, d), lambda j: (0, 0)),
        ],
        out_shape=[
            jax.ShapeDtypeStruct((1, 1), jnp.float32),
            jax.ShapeDtypeStruct((1, 1), jnp.int32),
            jax.ShapeDtypeStruct((1, 1), jnp.int32),
            jax.ShapeDtypeStruct((1, d), jnp.float32),
        ],
        scratch_shapes=[
            pltpu.VMEM((1, _NC), jnp.float32),
            pltpu.VMEM((1, _NC), jnp.int32),
        ],
    )(a16, cand, patch_lib)

    m_star = lax.dynamic_index_in_dim(patch_lib, staridx[0, 0], axis=0)

    s, smap = pl.pallas_call(
        _k2_body,
        grid=(nb,),
        in_specs=[
            pl.BlockSpec((1, d), lambda j: (0, 0)),
            pl.BlockSpec((1, d), lambda j: (0, 0)),
            pl.BlockSpec(memory_space=pltpu.SMEM),
            pl.BlockSpec((1, _MP), lambda j: (0, 0)),
            pl.BlockSpec((_IMG, _MP), lambda j: (0, 0)),
            pl.BlockSpec((_IMG, _MP), lambda j: (0, 0)),
            pl.BlockSpec((_IMG, _MP), lambda j: (0, 0)),
            pl.BlockSpec((_BN, d), lambda j: (j, 0)),
        ],
        out_specs=[
            pl.BlockSpec(memory_space=pltpu.SMEM),
            pl.BlockSpec((_IMG, _IMG), lambda j: (0, 0)),
        ],
        out_shape=[
            jax.ShapeDtypeStruct((1, 1), jnp.float32),
            jax.ShapeDtypeStruct((_IMG, _IMG), jnp.float32),
        ],
        scratch_shapes=[pltpu.VMEM((8, n), jnp.float32)],
    )(m_star, mtest, sstar, minval, jnp.asarray(_B), jnp.asarray(_CH),
      jnp.asarray(_CL), patch_lib)

    return s.reshape(()), smap.reshape(1, 1, _IMG, _IMG)


# speedup vs baseline: 1.6225x; 1.3213x over previous
import numpy as np
import jax
import jax.numpy as jnp
from jax import lax
from jax.experimental import pallas as pl
from jax.experimental.pallas import tpu as pltpu

_IMG = 224
_FMAP = 26
_SIGMA = 4.0
_NQ = 676
_MP = 704
_BN = 2048
_NC = 16
_DIMS = (((1,), (1,)), ((), ()))


def _resize_blur_matrix() -> np.ndarray:
    scale = _IMG / _FMAP
    x = (np.arange(_IMG) + 0.5) / scale - 0.5
    j = np.arange(_FMAP)
    w = np.maximum(0.0, 1.0 - np.abs(x[:, None] - j[None, :]))
    w = w / w.sum(axis=1, keepdims=True)
    ks = 2 * int(4.0 * _SIGMA) + 1
    r = ks // 2
    t = np.arange(ks) - r
    k = np.exp(-0.5 * (t / _SIGMA) ** 2)
    k = k / k.sum()
    g = np.zeros((_IMG, _IMG))
    for i in range(_IMG):
        for dt in range(ks):
            p = i + dt - r
            if p < 0:
                p = -p
            if p > _IMG - 1:
                p = 2 * (_IMG - 1) - p
            g[i, p] += k[dt]
    return (g @ w).astype(np.float32)


_A = _resize_blur_matrix()
_B = np.zeros((_IMG, _MP), np.float32)
_C = np.zeros((_IMG, _MP), np.float32)
_B[:, :_NQ] = _A[:, np.arange(_NQ) // _FMAP]
_C[:, :_NQ] = _A[:, np.arange(_NQ) % _FMAP]
_BF16 = np.dtype(jnp.bfloat16.dtype)
_CH = _C.astype(_BF16)
_CL = (_C - _CH.astype(np.float32)).astype(_BF16)


def _split(x):
    hi = x.astype(jnp.bfloat16)
    lo = (x - hi.astype(jnp.float32)).astype(jnp.bfloat16)
    return hi, lo


def _dot3(xh, xl, yh, yl):
    kw = dict(preferred_element_type=jnp.float32)
    return (lax.dot_general(xh, yh, _DIMS, **kw)
            + (lax.dot_general(xh, yl, _DIMS, **kw)
               + lax.dot_general(xl, yh, _DIMS, **kw)))


def _row_sumsq(x):
    eh, el = _split(x * x)
    ones = jnp.ones((1, x.shape[1]), jnp.bfloat16)
    kw = dict(preferred_element_type=jnp.float32)
    return (lax.dot_general(ones, eh, _DIMS, **kw)
            + lax.dot_general(ones, el, _DIMS, **kw))


def _k1_body(patch_ref, ahs_ref, lib_ref,
             minval_ref, cand_ref, a16_ref, curmin_ref):
    j = pl.program_id(0)
    nb = pl.num_programs(0)
    b = lib_ref[...]
    ab = lax.dot_general(b.astype(jnp.bfloat16), ahs_ref[...], _DIMS,
                         preferred_element_type=jnp.float32)
    b2 = jnp.sum(b * b, axis=1, keepdims=True)
    bmin = jnp.min(ab + b2, axis=0, keepdims=True)

    @pl.when(j == 0)
    def _init():
        curmin_ref[...] = jnp.full(curmin_ref.shape, jnp.inf, jnp.float32)

    curmin_ref[...] = jnp.minimum(curmin_ref[...], bmin)

    @pl.when(j == nb - 1)
    def _fin():
        a2 = _row_sumsq(patch_ref[...])
        dist = jnp.sqrt(jnp.maximum(curmin_ref[...] + a2, 1e-12))
        minval_ref[...] = dist
        cols = lax.broadcasted_iota(jnp.int32, dist.shape, 1)
        mv = jnp.where(cols < _NQ, dist, -1.0)
        rows_a = lax.broadcasted_iota(jnp.int32, patch_ref.shape, 0)
        cols16 = lax.broadcasted_iota(jnp.int32, (1, _NC), 1)
        candvec = jnp.full((1, _NC), _MP, jnp.int32)
        for k in range(_NC):
            vmax = jnp.max(mv)
            idx = jnp.min(jnp.where(mv == vmax, cols, _MP))
            candvec = jnp.where(cols16 == k, idx, candvec)
            mv = jnp.where(cols == idx, -1.0, mv)
            a16_ref[k:k + 1, :] = jnp.sum(
                jnp.where(rows_a == idx, patch_ref[...], 0.0),
                axis=0, keepdims=True)
        cand_ref[...] = candvec


def _k1b_body(a16_ref, cand_ref, lib_ref,
              sstar_ref, sidx_ref, staridx_ref, mtest_ref,
              curmin_ref, curidx_ref):
    j = pl.program_id(0)
    nb = pl.num_programs(0)
    a16 = a16_ref[...]
    ah, al = _split(a16)
    b = lib_ref[...]
    bh, bl = _split(b)
    ab = _dot3(bh, bl, ah, al)
    b2 = jnp.sum(b * b, axis=1, keepdims=True)
    d2 = b2 - 2.0 * ab
    bmin = jnp.min(d2, axis=0, keepdims=True)
    rows = lax.broadcasted_iota(jnp.int32, d2.shape, 0)
    bidx = jnp.min(jnp.where(d2 == bmin, rows, d2.shape[0]),
                   axis=0, keepdims=True)

    @pl.when(j == 0)
    def _init():
        curmin_ref[...] = jnp.full(curmin_ref.shape, jnp.inf, jnp.float32)
        curidx_ref[...] = jnp.zeros(curidx_ref.shape, jnp.int32)

    cm = curmin_ref[...]
    better = bmin < cm
    curmin_ref[...] = jnp.where(better, bmin, cm)
    curidx_ref[...] = jnp.where(better, j * _BN + bidx, curidx_ref[...])

    @pl.when(j == nb - 1)
    def _fin():
        a2 = _row_sumsq(a16)
        dist = jnp.sqrt(jnp.maximum(curmin_ref[...] + a2, 1e-12))
        cand = cand_ref[...]
        cols = lax.broadcasted_iota(jnp.int32, dist.shape, 1)
        smax = jnp.max(dist)
        sidx = jnp.min(jnp.where(dist == smax, cand, jnp.int32(2 ** 30)))
        lpos = jnp.min(jnp.where(cand == sidx, cols, _NC))
        sstar_ref[0, 0] = smax
        sidx_ref[0, 0] = sidx
        staridx_ref[0, 0] = jnp.sum(
            jnp.where(cols == lpos, curidx_ref[...], 0))
        rows_a = lax.broadcasted_iota(jnp.int32, a16.shape, 0)
        mtest_ref[...] = jnp.sum(
            jnp.where(rows_a == lpos, a16, 0.0), axis=0, keepdims=True)


def _k2_body(m8_ref, sstar_ref, minval_ref, b_ref, ch_ref,
             cl_ref, lib_ref, s_ref, smap_ref, d2all_ref):
    j = pl.program_id(0)
    nb = pl.num_programs(0)
    m = m8_ref[...]
    mh, ml = _split(m)
    b = lib_ref[...]
    bh, bl = _split(b)
    ab = _dot3(mh, ml, bh, bl)
    b2c = jnp.sum(b * b, axis=1, keepdims=True)
    b2 = lax.transpose(b2c, (1, 0))
    d2all_ref[:, pl.ds(j * _BN, _BN)] = b2 - 2.0 * ab

    @pl.when(j == nb - 1)
    def _fin():
        n = d2all_ref.shape[1]
        m2 = jnp.sum(m * m, axis=1, keepdims=True)
        all_d2 = d2all_ref[...] + m2
        star = all_d2[0:1, :]
        test = all_d2[1:2, :]
        cols = lax.broadcasted_iota(jnp.int32, star.shape, 1)
        inf = jnp.float32(jnp.inf)
        i0 = jnp.min(jnp.where(star == jnp.min(star), cols, n))
        star1 = jnp.where(cols == i0, inf, star)
        i1 = jnp.min(jnp.where(star1 == jnp.min(star1), cols, n))
        star2 = jnp.where(cols == i1, inf, star1)
        i2 = jnp.min(jnp.where(star2 == jnp.min(star2), cols, n))
        dk1 = jnp.sqrt(jnp.maximum(
            jnp.sum(jnp.where(cols == i1, test, 0.0)), 0.0))
        dk2 = jnp.sqrt(jnp.maximum(
            jnp.sum(jnp.where(cols == i2, test, 0.0)), 0.0))
        dcap = jnp.sqrt(jnp.float32(m.shape[1]))
        sstar = sstar_ref[0, 0]
        w = 1.0 - jnp.exp(sstar / dcap) / (jnp.exp(dk1 / dcap)
                                           + jnp.exp(dk2 / dcap))
        s_ref[0, 0] = w * sstar
        t = b_ref[...] * minval_ref[...]
        th, tl = _split(t)
        smap_ref[...] = _dot3(th, tl, ch_ref[...], cl_ref[...])


def kernel(patch, patch_lib):
    n, d = patch_lib.shape
    nb = n // _BN
    patch_p = jnp.pad(patch, ((0, _MP - patch.shape[0]), (0, 0)))
    ahs = (-2.0 * patch_p).astype(jnp.bfloat16)

    minval, cand, a16 = pl.pallas_call(
        _k1_body,
        grid=(nb,),
        in_specs=[
            pl.BlockSpec((_MP, d), lambda j: (0, 0)),
            pl.BlockSpec((_MP, d), lambda j: (0, 0)),
            pl.BlockSpec((_BN, d), lambda j: (j, 0)),
        ],
        out_specs=[
            pl.BlockSpec((1, _MP), lambda j: (0, 0)),
            pl.BlockSpec((1, _NC), lambda j: (0, 0)),
            pl.BlockSpec((_NC, d), lambda j: (0, 0)),
        ],
        out_shape=[
            jax.ShapeDtypeStruct((1, _MP), jnp.float32),
            jax.ShapeDtypeStruct((1, _NC), jnp.int32),
            jax.ShapeDtypeStruct((_NC, d), jnp.float32),
        ],
        scratch_shapes=[pltpu.VMEM((1, _MP), jnp.float32)],
    )(patch_p, ahs, patch_lib)

    sstar, sidx, staridx, mtest = pl.pallas_call(
        _k1b_body,
        grid=(nb,),
        in_specs=[
            pl.BlockSpec((_NC, d), lambda j: (0, 0)),
            pl.BlockSpec((1, _NC), lambda j: (0, 0)),
            pl.BlockSpec((_BN, d), lambda j: (j, 0)),
        ],
        out_specs=[
            pl.BlockSpec(memory_space=pltpu.SMEM),
            pl.BlockSpec(memory_space=pltpu.SMEM),
            pl.BlockSpec(memory_space=pltpu.SMEM),
            pl.BlockSpec((1, d), lambda j: (0, 0)),
        ],
        out_shape=[
            jax.ShapeDtypeStruct((1, 1), jnp.float32),
            jax.ShapeDtypeStruct((1, 1), jnp.int32),
            jax.ShapeDtypeStruct((1, 1), jnp.int32),
            jax.ShapeDtypeStruct((1, d), jnp.float32),
        ],
        scratch_shapes=[
            pltpu.VMEM((1, _NC), jnp.float32),
            pltpu.VMEM((1, _NC), jnp.int32),
        ],
    )(a16, cand, patch_lib)

    m_star = lax.dynamic_index_in_dim(patch_lib, staridx[0, 0], axis=0)
    m8 = jnp.zeros((8, d), jnp.float32)
    m8 = lax.dynamic_update_slice(m8, m_star, (0, 0))
    m8 = lax.dynamic_update_slice(m8, mtest, (1, 0))

    s, smap = pl.pallas_call(
        _k2_body,
        grid=(nb,),
        in_specs=[
            pl.BlockSpec((8, d), lambda j: (0, 0)),
            pl.BlockSpec(memory_space=pltpu.SMEM),
            pl.BlockSpec((1, _MP), lambda j: (0, 0)),
            pl.BlockSpec((_IMG, _MP), lambda j: (0, 0)),
            pl.BlockSpec((_IMG, _MP), lambda j: (0, 0)),
            pl.BlockSpec((_IMG, _MP), lambda j: (0, 0)),
            pl.BlockSpec((_BN, d), lambda j: (j, 0)),
        ],
        out_specs=[
            pl.BlockSpec(memory_space=pltpu.SMEM),
            pl.BlockSpec((_IMG, _IMG), lambda j: (0, 0)),
        ],
        out_shape=[
            jax.ShapeDtypeStruct((1, 1), jnp.float32),
            jax.ShapeDtypeStruct((_IMG, _IMG), jnp.float32),
        ],
        scratch_shapes=[pltpu.VMEM((8, n), jnp.float32)],
    )(m8, sstar, minval, jnp.asarray(_B), jnp.asarray(_CH),
      jnp.asarray(_CL), patch_lib)

    return s.reshape(()), smap.reshape(1, 1, _IMG, _IMG)
